# baseline (speedup 1.0000x reference)
import functools

import jax
import jax.numpy as jnp
from jax import lax
from jax.experimental import pallas as pl
from jax.experimental.pallas import tpu as pltpu
from jax.experimental.pallas import tpu_sc as plsc

B = 16384
F = 26
V = 100000
D = 16
ND = 13

NC = 2
NS = 16
NW = NC * NS
RPW = B // NW
CH = 8
LPC = CH * F
TPC = LPC // 16
NCHUNK = RPW // CH
LKW = RPW * F
IDX_PAD = LKW + 16
E1_PAD = LKW + 16
XD_PAD = RPW * ND + 16


def _fm_body(idx_h, xd_h, wb_h, e1_h, e2_h, out_h,
             idx_v, row_v, e1_v, e2_v, xd_v, wb_v, redsum_v, o_v,
             sem1, sem2a, sem2b):
    wid = lax.axis_index("s") * NC + lax.axis_index("c")
    base = wid * RPW

    pltpu.sync_copy(idx_h.at[pl.ds(wid * LKW, LKW)],
                    idx_v.at[pl.ds(0, LKW)])
    pltpu.sync_copy(xd_h.at[pl.ds(base * ND, RPW * ND)],
                    xd_v.at[pl.ds(0, RPW * ND)])
    pltpu.sync_copy(wb_h, wb_v)

    def rowinit(k, carry):
        row_v[pl.ds(k * 16, 16)] = lax.shift_right_logical(
            idx_v[pl.ds(k * 16, 16)], 3)
        return carry
    lax.fori_loop(0, LKW // 16, rowinit, 0)

    zero16 = jnp.zeros((16,), jnp.float32)
    e1_v[pl.ds(LKW, 16)] = zero16
    xd_v[pl.ds(RPW * ND, 16)] = zero16
    idx_v[pl.ds(LKW, 16)] = jnp.zeros((16,), jnp.int32)

    def fire2(ci, slot, sem):
        for t in range(TPC):
            p = ci * LPC + t * 16
            pltpu.async_copy(
                e2_h.at[row_v.at[pl.ds(p, 16)]],
                e2_v.at[slot, pl.ds(t * 16, 16)], sem)

    def drain2(slot, sem):
        for t in range(TPC):
            pltpu.make_async_copy(
                e2_h.at[row_v.at[pl.ds(0, 16)]],
                e2_v.at[slot, pl.ds(t * 16, 16)], sem).wait()

    fire2(0, 0, sem2a)

    def fire1(j, carry):
        pltpu.async_copy(e1_h.at[idx_v.at[pl.ds(j * 128, 128)]],
                         e1_v.at[pl.ds(j * 128, 128)], sem1)
        return carry
    lax.fori_loop(0, LKW // 128, fire1, 0)

    def drain1(j, carry):
        pltpu.make_async_copy(e1_h.at[idx_v.at[pl.ds(0, 128)]],
                              e1_v.at[pl.ds(0, 128)], sem1).wait()
        return carry
    lax.fori_loop(0, LKW // 128, drain1, 0)

    lane = jax.lax.iota(jnp.int32, 16)
    m10 = jnp.where(lane < F - 16, 1.0, 0.0).astype(jnp.float32)
    wvec = wb_v[...]
    wpad = jnp.where(lane < ND, wvec, 0.0).astype(jnp.float32)
    b0 = wvec[15]
    zvec = jnp.zeros((16,), jnp.float32)

    def chunkbody(ci, svec):
        slot = lax.rem(ci, 2)

        @pl.when(ci + 1 < NCHUNK)
        def _fire_next():
            @pl.when(slot == 0)
            def _():
                fire2(ci + 1, 1, sem2b)

            @pl.when(slot == 1)
            def _():
                fire2(ci + 1, 0, sem2a)

        @pl.when(slot == 0)
        def _():
            drain2(0, sem2a)

        @pl.when(slot == 1)
        def _():
            drain2(1, sem2b)

        lsel = slot * CH
        for r8 in range(CH):
            lr = ci * CH + r8
            fo = lr * F
            iA = idx_v[pl.ds(fo, 16)]
            iB = idx_v[pl.ds(fo + 16, 16)]
            oA = lax.shift_left(lax.bitwise_and(iA, 7), 4)
            oB = lax.shift_left(lax.bitwise_and(iB, 7), 4)
            acc = None
            acc2 = None
            for f in range(F):
                off = oA[f] if f < 16 else oB[f - 16]
                v = e2_v[slot, r8 * F + f, pl.ds(off, 16)]
                if acc is None:
                    acc, acc2 = v, v * v
                else:
                    acc = acc + v
                    acc2 = acc2 + v * v
            tot = 0.5 * (acc * acc - acc2)
            tot = tot + e1_v[pl.ds(fo, 16)]
            tot = tot + m10 * e1_v[pl.ds(fo + 16, 16)]
            tot = tot + wpad * xd_v[pl.ds(lr * ND, 16)]
            parts = [tot[d] for d in range(D)]
            while len(parts) > 1:
                parts = [parts[i] + parts[i + 1]
                         for i in range(0, len(parts), 2)]
            svec = jnp.where(lane == lsel + r8, parts[0] + b0, svec)

        @pl.when(slot == 1)
        def _store():
            redsum_v[pl.ds(ci * CH - CH, 16)] = svec
        return jnp.where(slot == 1, zvec, svec)
    lax.fori_loop(0, NCHUNK, chunkbody, zvec)

    def gbody(g, carry):
        v = redsum_v[pl.ds(g * 16, 16)]
        o_v[pl.ds(g * 16, 16)] = 1.0 / (1.0 + jnp.exp(-v))
        return carry
    lax.fori_loop(0, RPW // 16, gbody, 0)

    pltpu.sync_copy(o_v, out_h.at[pl.ds(base, RPW)])


@functools.partial(
    pl.kernel,
    mesh=plsc.VectorSubcoreMesh(core_axis_name="c", subcore_axis_name="s"),
    out_type=jax.ShapeDtypeStruct((B,), jnp.float32),
    scratch_types=[
        pltpu.VMEM((IDX_PAD,), jnp.int32),
        pltpu.VMEM((LKW,), jnp.int32),
        pltpu.VMEM((E1_PAD,), jnp.float32),
        pltpu.VMEM((2, LPC, 128), jnp.float32),
        pltpu.VMEM((XD_PAD,), jnp.float32),
        pltpu.VMEM((16,), jnp.float32),
        pltpu.VMEM((RPW,), jnp.float32),
        pltpu.VMEM((RPW,), jnp.float32),
        pltpu.SemaphoreType.DMA,
        pltpu.SemaphoreType.DMA,
        pltpu.SemaphoreType.DMA,
    ],
)
def _fm_kernel(idx_h, xd_h, wb_h, e1_h, e2_h, out_h, *scratch):
    _fm_body(idx_h, xd_h, wb_h, e1_h, e2_h, out_h, *scratch)


def kernel(X_sparse, X_dense, emb1_tables, emb2_tables, W_dense, b_dense):
    off = jnp.tile(jnp.arange(F, dtype=jnp.int32) * V, B)
    idx = X_sparse.reshape(B * F) + off
    xd = X_dense.reshape(B * ND)
    wb = jnp.concatenate([
        W_dense[:, 0],
        jnp.zeros((2,), jnp.float32),
        b_dense,
    ])
    e1 = emb1_tables[:, :, 0].reshape(F * V)
    e2 = emb2_tables.reshape(F * V // 8, 128)
    out = _fm_kernel(idx, xd, wb, e1, e2)
    return out.reshape(B, 1)


# speedup vs baseline: 1.0730x; 1.0730x over previous
import functools

import jax
import jax.numpy as jnp
from jax import lax
from jax.experimental import pallas as pl
from jax.experimental.pallas import tpu as pltpu
from jax.experimental.pallas import tpu_sc as plsc

B = 16384
F = 26
V = 100000
D = 16
ND = 13

NC = 2
NS = 16
NW = NC * NS
RPW = B // NW
CH = 64
TPC = CH * F // 128
NCHUNK = RPW // CH
LKW = RPW * F
IDX_PAD = LKW + 16
E1_PAD = LKW + 16
XD_PAD = RPW * ND + 16


def _fm_body(idx_h, xd_h, wb_h, e1_h, e2_h, out_h,
             idx_v, e1_v, e2_v, xd_v, wb_v, redsum_v, o_v,
             sem1, sem2a, sem2b):
    wid = lax.axis_index("s") * NC + lax.axis_index("c")
    base = wid * RPW

    pltpu.sync_copy(idx_h.at[pl.ds(wid * LKW, LKW)],
                    idx_v.at[pl.ds(0, LKW)])
    pltpu.sync_copy(xd_h.at[pl.ds(base * ND, RPW * ND)],
                    xd_v.at[pl.ds(0, RPW * ND)])
    pltpu.sync_copy(wb_h, wb_v)

    zero16 = jnp.zeros((16,), jnp.float32)
    e1_v[pl.ds(LKW, 16)] = zero16
    xd_v[pl.ds(RPW * ND, 16)] = zero16

    def fire2(ci, slot, sem):
        for t in range(TPC):
            p = ci * (CH * F) + t * 128
            pltpu.async_copy(
                e2_h.at[idx_v.at[pl.ds(p, 128)]],
                e2_v.at[slot, pl.ds(t * 128, 128)], sem)

    def drain2(slot, sem):
        for t in range(TPC):
            pltpu.make_async_copy(
                e2_h.at[idx_v.at[pl.ds(0, 128)]],
                e2_v.at[slot, pl.ds(t * 128, 128)], sem).wait()

    fire2(0, 0, sem2a)

    def fire1(j, carry):
        pltpu.async_copy(e1_h.at[idx_v.at[pl.ds(j * 128, 128)]],
                         e1_v.at[pl.ds(j * 128, 128)], sem1)
        return carry
    lax.fori_loop(0, LKW // 128, fire1, 0)

    def drain1(j, carry):
        pltpu.make_async_copy(e1_h.at[idx_v.at[pl.ds(0, 128)]],
                              e1_v.at[pl.ds(0, 128)], sem1).wait()
        return carry
    lax.fori_loop(0, LKW // 128, drain1, 0)

    lane = jax.lax.iota(jnp.int32, 16)
    m10 = jnp.where(lane < F - 16, 1.0, 0.0).astype(jnp.float32)
    wvec = wb_v[...]
    wpad = jnp.where(lane < ND, wvec, 0.0).astype(jnp.float32)
    b0 = wvec[15]

    def chunkbody(ci, carry):
        slot = lax.rem(ci, 2)

        @pl.when(ci + 1 < NCHUNK)
        def _fire_next():
            @pl.when(slot == 0)
            def _():
                fire2(ci + 1, 1, sem2b)

            @pl.when(slot == 1)
            def _():
                fire2(ci + 1, 0, sem2a)

        @pl.when(slot == 0)
        def _():
            drain2(0, sem2a)

        @pl.when(slot == 1)
        def _():
            drain2(1, sem2b)

        def groupbody(g, carry2):
            svec = jnp.zeros((16,), jnp.float32)
            for r16 in range(16):
                r = g * 16 + r16
                rb = r * F
                acc = e2_v[slot, rb, :]
                acc2 = acc * acc
                for f in range(1, F):
                    v = e2_v[slot, rb + f, :]
                    acc = acc + v
                    acc2 = acc2 + v * v
                tot = 0.5 * (acc * acc - acc2)
                lr = ci * CH + r
                fo = lr * F
                tot = tot + e1_v[pl.ds(fo, 16)]
                tot = tot + m10 * e1_v[pl.ds(fo + 16, 16)]
                tot = tot + wpad * xd_v[pl.ds(lr * ND, 16)]
                parts = [tot[d] for d in range(D)]
                while len(parts) > 1:
                    parts = [parts[i] + parts[i + 1]
                             for i in range(0, len(parts), 2)]
                svec = jnp.where(lane == r16, parts[0] + b0, svec)
            redsum_v[pl.ds(ci * CH + g * 16, 16)] = svec
            return carry2
        lax.fori_loop(0, CH // 16, groupbody, 0)
        return carry
    lax.fori_loop(0, NCHUNK, chunkbody, 0)

    def gbody(g, carry):
        v = redsum_v[pl.ds(g * 16, 16)]
        o_v[pl.ds(g * 16, 16)] = 1.0 / (1.0 + jnp.exp(-v))
        return carry
    lax.fori_loop(0, RPW // 16, gbody, 0)

    pltpu.sync_copy(o_v, out_h.at[pl.ds(base, RPW)])


@functools.partial(
    pl.kernel,
    mesh=plsc.VectorSubcoreMesh(core_axis_name="c", subcore_axis_name="s"),
    out_type=jax.ShapeDtypeStruct((B,), jnp.float32),
    compiler_params=pltpu.CompilerParams(use_tc_tiling_on_sc=False),
    scratch_types=[
        pltpu.VMEM((IDX_PAD,), jnp.int32),
        pltpu.VMEM((E1_PAD,), jnp.float32),
        pltpu.VMEM((2, CH * F, D), jnp.float32),
        pltpu.VMEM((XD_PAD,), jnp.float32),
        pltpu.VMEM((16,), jnp.float32),
        pltpu.VMEM((RPW,), jnp.float32),
        pltpu.VMEM((RPW,), jnp.float32),
        pltpu.SemaphoreType.DMA,
        pltpu.SemaphoreType.DMA,
        pltpu.SemaphoreType.DMA,
    ],
)
def _fm_kernel(idx_h, xd_h, wb_h, e1_h, e2_h, out_h, *scratch):
    _fm_body(idx_h, xd_h, wb_h, e1_h, e2_h, out_h, *scratch)


def kernel(X_sparse, X_dense, emb1_tables, emb2_tables, W_dense, b_dense):
    off = jnp.tile(jnp.arange(F, dtype=jnp.int32) * V, B)
    idx = X_sparse.reshape(B * F) + off
    xd = X_dense.reshape(B * ND)
    wb = jnp.concatenate([
        W_dense[:, 0],
        jnp.zeros((2,), jnp.float32),
        b_dense,
    ])
    e1 = emb1_tables.reshape(F * V) + jnp.float32(0.0)
    e2 = emb2_tables.reshape(F * V, D) + jnp.float32(0.0)
    out = _fm_kernel(idx, xd, wb, e1, e2)
    return out.reshape(B, 1)
